# baseline (speedup 1.0000x reference)
import functools

import jax
import jax.numpy as jnp
import numpy as np
from jax import lax
from jax.experimental import pallas as pl
from jax.experimental.pallas import tpu as pltpu
from jax.experimental.pallas import tpu_sc as plsc

_VOCAB = 100000
_PAD = _VOCAB
_D = 64
_DP = 128
_L = 16


@functools.cache
def _make_kernel(b, s, w):
    info = plsc.get_sparse_core_info()
    nc, ns = info.num_cores, info.num_subcores
    nw = nc * ns
    bpw = b // nw

    mesh = plsc.VectorSubcoreMesh(core_axis_name="c", subcore_axis_name="s")

    @functools.partial(
        pl.kernel,
        mesh=mesh,
        out_type=jax.ShapeDtypeStruct((nw, bpw, s, _D), jnp.bfloat16),
        scratch_types=[
            pltpu.VMEM((s, w, bpw), jnp.int32),
            pltpu.VMEM((s, bpw, _DP), jnp.bfloat16),
            pltpu.VMEM((2, bpw, _D), jnp.bfloat16),
            pltpu.VMEM((2 * _L, 2 * _L), jnp.bfloat16),
            pltpu.SemaphoreType.DMA,
            pltpu.SemaphoreType.DMA,
        ],
        compiler_params=pltpu.CompilerParams(use_tc_tiling_on_sc=False),
    )
    def k(idx_hbm, table_hbm, lut_hbm, out_hbm, idx_v, acc_v, stage_v, lut_v,
          sem, sem_out):
        wid = lax.axis_index("s") * nc + lax.axis_index("c")
        b0 = wid * bpw
        pltpu.sync_copy(idx_hbm.at[:, :, pl.ds(b0, bpw)], idx_v)
        pltpu.sync_copy(lut_hbm, lut_v)

        zero = jnp.zeros((2 * _L,), jnp.bfloat16)

        def fire_body(si, carry):
            for bl in range(bpw):
                for d in range(_DP // (2 * _L)):
                    acc_v[si, bl, pl.ds(d * 2 * _L, 2 * _L)] = zero
            for wj in range(w):
                pltpu.async_copy(
                    table_hbm.at[idx_v.at[si, wj]], acc_v.at[si], sem,
                    add=True)
            return carry

        lax.fori_loop(0, s, fire_body, 0)

        def drain_body(si, carry):
            for wj in range(w):
                pltpu.make_async_copy(
                    table_hbm.at[idx_v.at[0, 0]], acc_v.at[0], sem).wait()
            return carry

        lax.fori_loop(0, s, drain_body, 0)

        def scale_body(si, carry):
            @pl.when(si >= 2)
            def _():
                pltpu.make_async_copy(
                    stage_v.at[0], out_hbm.at[wid, :, 0], sem_out).wait()

            cnts = []
            for bh in range(bpw // _L):
                cnt = jnp.zeros((_L,), jnp.int32)
                for wj in range(w):
                    v = idx_v[si, wj, pl.ds(bh * _L, _L)]
                    cnt = cnt + jnp.where(v != _PAD, 1, 0)
                cnts.append(cnt)
            par = si % 2
            for bl in range(bpw):
                c = cnts[bl // _L][bl % _L]
                scb = lut_v[c, pl.ds(0, 2 * _L)]
                for d in range(_D // (2 * _L)):
                    sl = pl.ds(d * 2 * _L, 2 * _L)
                    stage_v[par, bl, sl] = acc_v[si, bl, sl] * scb
            pltpu.async_copy(stage_v.at[par], out_hbm.at[wid, :, si], sem_out)
            return carry

        lax.fori_loop(0, s, scale_body, 0)

        def out_drain_body(i, carry):
            pltpu.make_async_copy(
                stage_v.at[0], out_hbm.at[wid, :, 0], sem_out).wait()
            return carry

        lax.fori_loop(0, 2, out_drain_body, 0)

    return k


def kernel(inpt, weights):
    b, s, w = inpt.shape
    idx = jnp.transpose(inpt.astype(jnp.int32), (1, 2, 0))
    wb = jnp.pad(weights.astype(jnp.bfloat16), ((0, 0), (0, _DP - _D)))
    inv = 1.0 / np.maximum(np.arange(32, dtype=np.float32), 1.0)
    lut = jnp.asarray(
        np.repeat(inv[:, None], 32, axis=1), dtype=jnp.bfloat16)
    out = _make_kernel(b, s, w)(idx, wb, lut)
    return out.astype(jnp.float32).reshape(b, s, _D)


# speedup vs baseline: 1.7852x; 1.7852x over previous
import functools

import jax
import jax.numpy as jnp
from jax import lax
from jax.experimental import pallas as pl
from jax.experimental.pallas import tpu as pltpu
from jax.experimental.pallas import tpu_sc as plsc

_VOCAB = 100000
_PAD = _VOCAB
_D = 64
_L = 16


@functools.cache
def _make_kernel(b, s, w):
    info = plsc.get_sparse_core_info()
    nc, ns = info.num_cores, info.num_subcores
    nw = nc * ns
    bpw = b // nw
    nd = _D // _L

    mesh = plsc.VectorSubcoreMesh(core_axis_name="c", subcore_axis_name="s")

    @functools.partial(
        pl.kernel,
        mesh=mesh,
        out_type=jax.ShapeDtypeStruct((nw, bpw, s, _D), jnp.float32),
        scratch_types=[
            pltpu.VMEM((s, w, bpw), jnp.int32),
            pltpu.VMEM((s, bpw, _D), jnp.float32),
            pltpu.SemaphoreType.DMA,
            pltpu.SemaphoreType.DMA,
        ],
        compiler_params=pltpu.CompilerParams(use_tc_tiling_on_sc=False),
    )
    def k(idx_hbm, table_hbm, out_hbm, idx_v, acc_v, sem, sem_out):
        wid = lax.axis_index("s") * nc + lax.axis_index("c")
        b0 = wid * bpw
        pltpu.sync_copy(idx_hbm.at[:, :, pl.ds(b0, bpw)], idx_v)

        zero = jnp.zeros((_L,), jnp.float32)

        def fire_body(si, carry):
            for bl in range(bpw):
                for d in range(nd):
                    acc_v[si, bl, pl.ds(d * _L, _L)] = zero
            for wj in range(w):
                pltpu.async_copy(
                    table_hbm.at[idx_v.at[si, wj]], acc_v.at[si], sem,
                    add=True)
            return carry

        lax.fori_loop(0, s, fire_body, 0)

        def drain_body(si, carry):
            for wj in range(w):
                pltpu.make_async_copy(
                    table_hbm.at[idx_v.at[0, 0]], acc_v.at[0], sem).wait()
            return carry

        lax.fori_loop(0, s, drain_body, 0)

        def scale_body(si, carry):
            invs = []
            for bh in range(bpw // _L):
                cnt = jnp.zeros((_L,), jnp.int32)
                for wj in range(w):
                    v = idx_v[si, wj, pl.ds(bh * _L, _L)]
                    cnt = cnt + jnp.where(v != _PAD, 1, 0)
                invs.append(1.0 / jnp.maximum(cnt, 1).astype(jnp.float32))
            for bl in range(bpw):
                sc = invs[bl // _L][bl % _L]
                for d in range(nd):
                    scaled = acc_v[si, bl, pl.ds(d * _L, _L)] * sc
                    acc_v[si, bl, pl.ds(d * _L, _L)] = scaled
            pltpu.async_copy(acc_v.at[si], out_hbm.at[wid, :, si], sem_out)
            return carry

        lax.fori_loop(0, s, scale_body, 0)

        def out_drain_body(si, carry):
            pltpu.make_async_copy(
                acc_v.at[0], out_hbm.at[wid, :, 0], sem_out).wait()
            return carry

        lax.fori_loop(0, s, out_drain_body, 0)

    return k


def kernel(inpt, weights):
    b, s, w = inpt.shape
    nw = 32
    bpw = b // nw
    idx = jnp.transpose(inpt.astype(jnp.int32), (1, 2, 0))
    out = _make_kernel(b, s, w)(idx, weights)
    return out.reshape(b, s, _D)
